# baseline (speedup 1.0000x reference)
import functools

import jax
import jax.numpy as jnp
from jax import lax
from jax.experimental import pallas as pl
from jax.experimental.pallas import tpu as pltpu
from jax.experimental.pallas import tpu_sc as plsc

N = 10000
E = 320000
IN_DIM = 128
D = 64
DW = 64
NG = 128
NCLS = 10

_NS = 16

_K = 128
_NB = 4
_NCH = 160
_E_PAD = _NS * _K * _NCH
_N_PAD = 10112
_RPT = _N_PAD // _NS


def _edge_agg_body(y_hbm, src_hbm, dst_hbm, zeros_hbm, out_hbm,
                   src_v, dst_v, rows, sems, acc_sh):
    sid = lax.axis_index("s")
    pltpu.sync_copy(zeros_hbm.at[pl.ds(sid * _RPT, _RPT)],
                    acc_sh.at[pl.ds(sid * _RPT, _RPT)])
    pltpu.sync_copy(src_hbm.at[sid], src_v)
    pltpu.sync_copy(dst_hbm.at[sid], dst_v)
    plsc.subcore_barrier()

    for b in range(_NB):
        pltpu.async_copy(y_hbm.at[src_v.at[b]], rows[b], sems[b])

    @pl.loop(0, _NCH - _NB, step=_NB)
    def chunks(g):
        for b in range(_NB):
            j = g + b
            pltpu.make_async_copy(y_hbm.at[src_v.at[j]], rows[b],
                                  sems[b]).wait()
            pltpu.sync_copy(rows[b], acc_sh.at[dst_v.at[j]], add=True)
            pltpu.async_copy(y_hbm.at[src_v.at[j + _NB]], rows[b], sems[b])

    for b in range(_NB):
        j = _NCH - _NB + b
        pltpu.make_async_copy(y_hbm.at[src_v.at[j]], rows[b], sems[b]).wait()
        pltpu.sync_copy(rows[b], acc_sh.at[dst_v.at[j]], add=True)

    plsc.subcore_barrier()
    pltpu.sync_copy(acc_sh.at[pl.ds(sid * _RPT, _RPT)],
                    out_hbm.at[pl.ds(sid * _RPT, _RPT)])


@functools.cache
def _make_edge_agg():
    return pl.kernel(
        _edge_agg_body,
        out_type=jax.ShapeDtypeStruct((_N_PAD, DW), jnp.float32),
        mesh=plsc.VectorSubcoreMesh(core_axis_name="c", subcore_axis_name="s",
                                    num_cores=1),
        compiler_params=pltpu.CompilerParams(use_tc_tiling_on_sc=False),
        scratch_types=[
            pltpu.VMEM((_NCH, _K), jnp.int32),
            pltpu.VMEM((_NCH, _K), jnp.int32),
            tuple(pltpu.VMEM((_K, DW), jnp.float32) for _ in range(_NB)),
            tuple(pltpu.SemaphoreType.DMA for _ in range(_NB)),
            pltpu.VMEM_SHARED((_N_PAD, DW), jnp.float32),
        ],
    )


def _edge_agg(y, srcp, dstp, zeros):
    return _make_edge_agg()(y, srcp, dstp, zeros)


def _tc1_body(x_ref, w_ref, o_ref):
    o_ref[...] = jnp.dot(x_ref[...], w_ref[...],
                         preferred_element_type=jnp.float32)


def _tc2_body(y0_ref, p0_ref, b0_ref, sc_ref, sh_ref, w01_ref,
              b01_ref, w10_ref, o_ref):
    pre = (y0_ref[...] + p0_ref[...]) + b0_ref[...]
    t = jnp.maximum(pre * sc_ref[...] + sh_ref[...], 0.0)
    h = jnp.dot(t, w01_ref[...], preferred_element_type=jnp.float32)
    h = jnp.maximum(h + b01_ref[...], 0.0)
    o_ref[...] = jnp.dot(h, w10_ref[...], preferred_element_type=jnp.float32)


def _tc3_body(y1_ref, p0_ref, b1_ref, sc_ref, sh_ref, w11_ref,
              b11_ref, bat_ref, wc_ref, bc_ref, o_ref):
    pre = (y1_ref[...] + p0_ref[...]) + b1_ref[...]
    v = jnp.maximum(pre * sc_ref[...] + sh_ref[...], 0.0)
    h2 = jnp.dot(v, w11_ref[...], preferred_element_type=jnp.float32)
    h2 = h2 + b11_ref[...]
    gids = lax.broadcasted_iota(jnp.int32, (NG, _N_PAD), 0)
    pt = (gids == bat_ref[...]).astype(jnp.float32)
    pooled = jnp.dot(pt, h2, preferred_element_type=jnp.float32)
    cnt = jnp.sum(pt, axis=1, keepdims=True)
    pooled = pooled / jnp.maximum(cnt, 1.0)
    o_ref[...] = jnp.dot(pooled, wc_ref[...],
                         preferred_element_type=jnp.float32) + bc_ref[...]


_tc1 = pl.pallas_call(
    _tc1_body,
    out_shape=jax.ShapeDtypeStruct((_N_PAD, D), jnp.float32))
_tc2 = pl.pallas_call(
    _tc2_body,
    out_shape=jax.ShapeDtypeStruct((_N_PAD, D), jnp.float32))
_tc3 = pl.pallas_call(
    _tc3_body,
    out_shape=jax.ShapeDtypeStruct((NG, NCLS), jnp.float32))


def _bn_fold(g, b, rm, rv, eps=1e-5):
    scale = g * lax.rsqrt(rv + eps)
    shift = b - rm * scale
    return scale.reshape(1, D), shift.reshape(1, D)


def kernel(x, edge_index, batch, c0_l0_W, c0_l0_b, c0_bn_g, c0_bn_b,
           c0_bn_rm, c0_bn_rv, c0_l1_W, c0_l1_b, c1_l0_W, c1_l0_b, c1_bn_g,
           c1_bn_b, c1_bn_rm, c1_bn_rv, c1_l1_W, c1_l1_b, cls_W, cls_b):
    f32 = jnp.float32
    xp = jnp.concatenate([x, jnp.zeros((_N_PAD - N, IN_DIM), f32)], axis=0)
    pad_idx = jnp.full((_E_PAD - E,), N, jnp.int32)
    srcp = jnp.concatenate([edge_index[0], pad_idx]).reshape(_NS, _NCH, _K)
    dstp = jnp.concatenate([edge_index[1], pad_idx]).reshape(_NS, _NCH, _K)
    batp = jnp.concatenate(
        [batch, jnp.full((_N_PAD - N,), -1, jnp.int32)]).reshape(1, _N_PAD)
    zeros = jnp.zeros((_N_PAD, DW), f32)
    sc0, sh0 = _bn_fold(c0_bn_g, c0_bn_b, c0_bn_rm, c0_bn_rv)
    sc1, sh1 = _bn_fold(c1_bn_g, c1_bn_b, c1_bn_rm, c1_bn_rv)

    y0 = _tc1(xp, c0_l0_W.T)
    agg0 = _edge_agg(y0, srcp, dstp, zeros)
    y1 = _tc2(y0, agg0, c0_l0_b.reshape(1, D), sc0, sh0,
              c0_l1_W.T, c0_l1_b.reshape(1, D), c1_l0_W.T)
    agg1 = _edge_agg(y1, srcp, dstp, zeros)
    out = _tc3(y1, agg1, c1_l0_b.reshape(1, D), sc1, sh1,
               c1_l1_W.T, c1_l1_b.reshape(1, D), batp,
               cls_W.T, cls_b.reshape(1, NCLS))
    return out


# speedup vs baseline: 1.2611x; 1.2611x over previous
import functools

import jax
import jax.numpy as jnp
from jax import lax
from jax.experimental import pallas as pl
from jax.experimental.pallas import tpu as pltpu
from jax.experimental.pallas import tpu_sc as plsc

N = 10000
E = 320000
IN_DIM = 128
D = 64
DW = 64
NG = 128
NCLS = 10

_NC = 2
_NS = 16
_NW = _NC * _NS

_K = 128
_NB = 8
_NCH = 80
_E_PAD = _NW * _K * _NCH
_N_PAD = 10112
_RPT = _N_PAD // _NS


def _edge_agg_body(y_hbm, src_hbm, dst_hbm, zeros_hbm, out_hbm,
                   src_v, dst_v, rows, sems, acc_sh):
    cid = lax.axis_index("c")
    sid = lax.axis_index("s")
    wid = sid * _NC + cid
    pltpu.sync_copy(zeros_hbm.at[pl.ds(sid * _RPT, _RPT)],
                    acc_sh.at[pl.ds(sid * _RPT, _RPT)])
    pltpu.sync_copy(src_hbm.at[wid], src_v)
    pltpu.sync_copy(dst_hbm.at[wid], dst_v)
    plsc.subcore_barrier()

    for b in range(_NB):
        pltpu.async_copy(y_hbm.at[src_v.at[b]], rows[b], sems[b])

    @pl.loop(0, _NCH - _NB, step=_NB)
    def chunks(g):
        for b in range(_NB):
            j = g + b
            pltpu.make_async_copy(y_hbm.at[src_v.at[j]], rows[b],
                                  sems[b]).wait()
            pltpu.sync_copy(rows[b], acc_sh.at[dst_v.at[j]], add=True)
            pltpu.async_copy(y_hbm.at[src_v.at[j + _NB]], rows[b], sems[b])

    for b in range(_NB):
        j = _NCH - _NB + b
        pltpu.make_async_copy(y_hbm.at[src_v.at[j]], rows[b], sems[b]).wait()
        pltpu.sync_copy(rows[b], acc_sh.at[dst_v.at[j]], add=True)

    plsc.subcore_barrier()
    pltpu.sync_copy(acc_sh.at[pl.ds(sid * _RPT, _RPT)],
                    out_hbm.at[cid, pl.ds(sid * _RPT, _RPT)])


@functools.cache
def _make_edge_agg():
    return pl.kernel(
        _edge_agg_body,
        out_type=jax.ShapeDtypeStruct((_NC, _N_PAD, DW), jnp.float32),
        mesh=plsc.VectorSubcoreMesh(core_axis_name="c", subcore_axis_name="s"),
        compiler_params=pltpu.CompilerParams(use_tc_tiling_on_sc=False),
        scratch_types=[
            pltpu.VMEM((_NCH, _K), jnp.int32),
            pltpu.VMEM((_NCH, _K), jnp.int32),
            tuple(pltpu.VMEM((_K, DW), jnp.float32) for _ in range(_NB)),
            tuple(pltpu.SemaphoreType.DMA for _ in range(_NB)),
            pltpu.VMEM_SHARED((_N_PAD, DW), jnp.float32),
        ],
    )


def _edge_agg(y, srcp, dstp, zeros):
    return _make_edge_agg()(y, srcp, dstp, zeros)


def _tc1_body(x_ref, w_ref, o_ref):
    o_ref[...] = jnp.dot(x_ref[...], w_ref[...],
                         preferred_element_type=jnp.float32)


def _tc2_body(y0_ref, p0_ref, p1_ref, b0_ref, sc_ref, sh_ref, w01_ref,
              b01_ref, w10_ref, o_ref):
    pre = (y0_ref[...] + p0_ref[...] + p1_ref[...]) + b0_ref[...]
    t = jnp.maximum(pre * sc_ref[...] + sh_ref[...], 0.0)
    h = jnp.dot(t, w01_ref[...], preferred_element_type=jnp.float32)
    h = jnp.maximum(h + b01_ref[...], 0.0)
    o_ref[...] = jnp.dot(h, w10_ref[...], preferred_element_type=jnp.float32)


def _tc3_body(y1_ref, p0_ref, p1_ref, b1_ref, sc_ref, sh_ref, w11_ref,
              b11_ref, bat_ref, wc_ref, bc_ref, o_ref):
    pre = (y1_ref[...] + p0_ref[...] + p1_ref[...]) + b1_ref[...]
    v = jnp.maximum(pre * sc_ref[...] + sh_ref[...], 0.0)
    h2 = jnp.dot(v, w11_ref[...], preferred_element_type=jnp.float32)
    h2 = h2 + b11_ref[...]
    gids = lax.broadcasted_iota(jnp.int32, (NG, _N_PAD), 0)
    pt = (gids == bat_ref[...]).astype(jnp.float32)
    pooled = jnp.dot(pt, h2, preferred_element_type=jnp.float32)
    cnt = jnp.sum(pt, axis=1, keepdims=True)
    pooled = pooled / jnp.maximum(cnt, 1.0)
    o_ref[...] = jnp.dot(pooled, wc_ref[...],
                         preferred_element_type=jnp.float32) + bc_ref[...]


_tc1 = pl.pallas_call(
    _tc1_body,
    out_shape=jax.ShapeDtypeStruct((_N_PAD, DW), jnp.float32))
_tc2 = pl.pallas_call(
    _tc2_body,
    out_shape=jax.ShapeDtypeStruct((_N_PAD, DW), jnp.float32))
_tc3 = pl.pallas_call(
    _tc3_body,
    out_shape=jax.ShapeDtypeStruct((NG, NCLS), jnp.float32))


def _bn_fold(g, b, rm, rv, eps=1e-5):
    scale = g * lax.rsqrt(rv + eps)
    shift = b - rm * scale
    return scale.reshape(1, D), shift.reshape(1, D)


def kernel(x, edge_index, batch, c0_l0_W, c0_l0_b, c0_bn_g, c0_bn_b,
           c0_bn_rm, c0_bn_rv, c0_l1_W, c0_l1_b, c1_l0_W, c1_l0_b, c1_bn_g,
           c1_bn_b, c1_bn_rm, c1_bn_rv, c1_l1_W, c1_l1_b, cls_W, cls_b):
    f32 = jnp.float32
    xp = jnp.concatenate([x, jnp.zeros((_N_PAD - N, IN_DIM), f32)], axis=0)
    pad_idx = jnp.full((_E_PAD - E,), N, jnp.int32)
    srcp = jnp.concatenate([edge_index[0], pad_idx]).reshape(_NW, _NCH, _K)
    dstp = jnp.concatenate([edge_index[1], pad_idx]).reshape(_NW, _NCH, _K)
    batp = jnp.concatenate(
        [batch, jnp.full((_N_PAD - N,), -1, jnp.int32)]).reshape(1, _N_PAD)
    zeros = jnp.zeros((_N_PAD, DW), f32)
    sc0, sh0 = _bn_fold(c0_bn_g, c0_bn_b, c0_bn_rm, c0_bn_rv)
    sc1, sh1 = _bn_fold(c1_bn_g, c1_bn_b, c1_bn_rm, c1_bn_rv)

    y0 = _tc1(xp, c0_l0_W.T)
    parts0 = _edge_agg(y0, srcp, dstp, zeros)
    y1 = _tc2(y0, parts0[0], parts0[1], c0_l0_b.reshape(1, D), sc0, sh0,
              c0_l1_W.T, c0_l1_b.reshape(1, D), c1_l0_W.T)
    parts1 = _edge_agg(y1, srcp, dstp, zeros)
    out = _tc3(y1, parts1[0], parts1[1], c1_l0_b.reshape(1, D), sc1, sh1,
               c1_l1_W.T, c1_l1_b.reshape(1, D), batp,
               cls_W.T, cls_b.reshape(1, NCLS))
    return out
